# baseline (speedup 1.0000x reference)
import jax
import jax.numpy as jnp
from jax.experimental import pallas as pl
from jax.experimental.pallas import tpu as pltpu

_BN_EPS = 1e-5


def _stats_kernel(x_ref, w_ref, sum_ref, ssq_ref):
    x = x_ref[0].astype(jnp.bfloat16)
    y = jnp.dot(w_ref[...], x, preferred_element_type=jnp.float32)
    sum_ref[0] = jnp.sum(y, axis=1, keepdims=True)
    ssq_ref[0] = jnp.sum(y * y, axis=1, keepdims=True)


def _apply_kernel(x_ref, w_ref, shift_ref, o_ref):
    x32 = x_ref[0]
    y = jnp.dot(w_ref[...], x32.astype(jnp.bfloat16),
                preferred_element_type=jnp.float32)
    o_ref[0] = jnp.maximum(y + shift_ref[...], 0.0) + x32


def kernel(x, conv_w, conv_b, bn_gamma, bn_beta):
    del conv_b
    N, C_in, L = x.shape
    C_out = conv_w.shape[0]
    w32 = conv_w[:, :, 0].astype(jnp.float32)
    w16 = w32.astype(jnp.bfloat16)

    grid = (N,)
    x_spec = pl.BlockSpec((1, C_in, L), lambda n: (n, 0, 0))
    w_spec = pl.BlockSpec((C_out, C_in), lambda n: (0, 0))
    stat_spec = pl.BlockSpec((1, C_out, 1), lambda n: (n, 0, 0))

    psum, pssq = pl.pallas_call(
        _stats_kernel,
        out_shape=(jax.ShapeDtypeStruct((N, C_out, 1), jnp.float32),
                   jax.ShapeDtypeStruct((N, C_out, 1), jnp.float32)),
        grid=grid,
        in_specs=[x_spec, w_spec],
        out_specs=(stat_spec, stat_spec),
        compiler_params=pltpu.CompilerParams(
            dimension_semantics=("parallel",)),
    )(x, w16)

    r = N * L
    sum_y = jnp.sum(psum[:, :, 0], axis=0)
    ssq_y = jnp.sum(pssq[:, :, 0], axis=0)
    mean = sum_y / r
    var = jnp.maximum(ssq_y / r - mean * mean, 0.0)
    scale = bn_gamma * jax.lax.rsqrt(var + _BN_EPS)
    shift = (bn_beta - mean * scale).reshape(C_out, 1)
    w_scaled = (w32 * scale[:, None]).astype(jnp.bfloat16)

    vec_spec = pl.BlockSpec((C_out, 1), lambda n: (0, 0))
    out = pl.pallas_call(
        _apply_kernel,
        out_shape=jax.ShapeDtypeStruct((N, C_out, L), x.dtype),
        grid=grid,
        in_specs=[x_spec, w_spec, vec_spec],
        out_specs=pl.BlockSpec((1, C_out, L), lambda n: (n, 0, 0)),
        compiler_params=pltpu.CompilerParams(
            dimension_semantics=("parallel",)),
    )(x, w_scaled, shift)
    return out


# speedup vs baseline: 1.4252x; 1.4252x over previous
import jax
import jax.numpy as jnp
from jax.experimental import pallas as pl
from jax.experimental.pallas import tpu as pltpu

_BN_EPS = 1e-5


def _stats_kernel(x_ref, w_ref, sum_ref, ssq_ref, *, nb):
    @pl.when(pl.program_id(1) == 0)
    def _():
        sum_ref[...] = jnp.zeros_like(sum_ref)
        ssq_ref[...] = jnp.zeros_like(ssq_ref)

    w = w_ref[...]
    s = None
    q = None
    for b in range(nb):
        x = x_ref[b].astype(jnp.bfloat16)
        y = jnp.dot(w, x, preferred_element_type=jnp.float32)
        sb = jnp.sum(y, axis=1, keepdims=True)
        qb = jnp.sum(y * y, axis=1, keepdims=True)
        s = sb if s is None else s + sb
        q = qb if q is None else q + qb
    sum_ref[0] += s
    ssq_ref[0] += q


def _apply_kernel(x_ref, w_ref, shift_ref, o_ref, *, nb):
    w = w_ref[...]
    shift = shift_ref[...]
    for b in range(nb):
        x32 = x_ref[b]
        y = jnp.dot(w, x32.astype(jnp.bfloat16),
                    preferred_element_type=jnp.float32)
        o_ref[b] = jnp.maximum(y + shift, 0.0) + x32


def kernel(x, conv_w, conv_b, bn_gamma, bn_beta):
    del conv_b
    N, C_in, L = x.shape
    C_out = conv_w.shape[0]
    w32 = conv_w[:, :, 0].astype(jnp.float32)
    w16 = w32.astype(jnp.bfloat16)

    p = 2 if N % 2 == 0 else 1
    nb = next(b for b in (4, 2, 1) if N % (p * b) == 0)
    steps = N // (p * nb)

    w_spec = pl.BlockSpec((C_out, C_in), lambda *_: (0, 0))

    import functools
    stats_grid = (p, steps)
    x_spec0 = pl.BlockSpec((nb, C_in, L), lambda i, j: (i * steps + j, 0, 0))
    stat_spec = pl.BlockSpec((1, C_out, 1), lambda i, j: (i, 0, 0))
    psum, pssq = pl.pallas_call(
        functools.partial(_stats_kernel, nb=nb),
        out_shape=(jax.ShapeDtypeStruct((p, C_out, 1), jnp.float32),
                   jax.ShapeDtypeStruct((p, C_out, 1), jnp.float32)),
        grid=stats_grid,
        in_specs=[x_spec0, w_spec],
        out_specs=(stat_spec, stat_spec),
        compiler_params=pltpu.CompilerParams(
            dimension_semantics=("parallel", "arbitrary")),
    )(x, w16)

    r = N * L
    sum_y = jnp.sum(psum[:, :, 0], axis=0)
    ssq_y = jnp.sum(pssq[:, :, 0], axis=0)
    mean = sum_y / r
    var = jnp.maximum(ssq_y / r - mean * mean, 0.0)
    scale = bn_gamma * jax.lax.rsqrt(var + _BN_EPS)
    shift = (bn_beta - mean * scale).reshape(C_out, 1)
    w_scaled = (w32 * scale[:, None]).astype(jnp.bfloat16)

    grid1 = (N // nb,)
    x_spec1 = pl.BlockSpec((nb, C_in, L), lambda n: (n, 0, 0))
    vec_spec = pl.BlockSpec((C_out, 1), lambda n: (0, 0))
    out = pl.pallas_call(
        functools.partial(_apply_kernel, nb=nb),
        out_shape=jax.ShapeDtypeStruct((N, C_out, L), x.dtype),
        grid=grid1,
        in_specs=[x_spec1, w_spec, vec_spec],
        out_specs=pl.BlockSpec((nb, C_out, L), lambda n: (n, 0, 0)),
        compiler_params=pltpu.CompilerParams(
            dimension_semantics=("parallel",)),
    )(x, w_scaled, shift)
    return out
